# baseline (speedup 1.0000x reference)
import functools

import jax
import jax.numpy as jnp
from jax import lax
from jax.experimental import pallas as pl
from jax.experimental.pallas import tpu as pltpu
from jax.experimental.pallas import tpu_sc as plsc

S, H, E, I, K = 2048, 1024, 8, 1024, 2
ALPHA, LIMIT = 1.702, 7.0
M = 256
A = S * K
NB = A // M + E
P = NB * M
RB = 256


def _router_body(x_ref, rw_ref, rb_ref, tw_ref, ti_ref):
    x = x_ref[...]
    rw = rw_ref[...]
    logits = lax.dot_general(x, rw, (((1,), (1,)), ((), ())),
                             preferred_element_type=jnp.float32) + rb_ref[...]
    iota = lax.broadcasted_iota(jnp.int32, (RB, E), 1)
    m0 = jnp.max(logits, axis=1, keepdims=True)
    i0 = jnp.min(jnp.where(logits == m0, iota, E), axis=1, keepdims=True)
    masked = jnp.where(iota == i0, -jnp.inf, logits)
    m1 = jnp.max(masked, axis=1, keepdims=True)
    i1 = jnp.min(jnp.where(masked == m1, iota, E), axis=1, keepdims=True)
    e1 = jnp.exp(m1 - m0)
    w0 = 1.0 / (1.0 + e1)
    w1 = e1 * w0
    lane2 = lax.broadcasted_iota(jnp.int32, (RB, 2), 1)
    tw_ref[...] = jnp.where(lane2 == 0, w0, w1)
    ti_ref[...] = jnp.where(lane2 == 0, i0, i1)


_router = pl.pallas_call(
    _router_body,
    grid=(S // RB,),
    in_specs=[
        pl.BlockSpec((RB, H), lambda b: (b, 0)),
        pl.BlockSpec((E, H), lambda b: (0, 0)),
        pl.BlockSpec((1, E), lambda b: (0, 0)),
    ],
    out_specs=[
        pl.BlockSpec((RB, 2), lambda b: (b, 0)),
        pl.BlockSpec((RB, 2), lambda b: (b, 0)),
    ],
    out_shape=[
        jax.ShapeDtypeStruct((S, 2), jnp.float32),
        jax.ShapeDtypeStruct((S, 2), jnp.int32),
    ],
)


def _mlp_body(be_ref, bidx_ref, x_ref, wgu_ref, wd_ref,
              bgu_ref, bd_ref, y_ref, p_ref):
    del be_ref, bidx_ref
    @pl.when(pl.program_id(0) == 0)
    def _build_selector():
        r = lax.broadcasted_iota(jnp.int32, (2 * I, I), 0)
        c = lax.broadcasted_iota(jnp.int32, (2 * I, I), 1)
        p_ref[...] = (r == 2 * c).astype(jnp.bfloat16)

    x = x_ref[...].astype(jnp.bfloat16)
    wgu = wgu_ref[0].astype(jnp.bfloat16)
    gu = jnp.dot(x, wgu, preferred_element_type=jnp.float32) + bgu_ref[0]
    gate = jnp.minimum(gu, LIMIT)
    glu = gate * jax.nn.sigmoid(gate * ALPHA)
    up1 = jnp.clip(gu, -LIMIT, LIMIT) + 1.0
    up1s = pltpu.roll(up1, 2 * I - 1, 1)
    prod = (glu * up1s).astype(jnp.bfloat16)
    act = jnp.dot(prod, p_ref[...],
                  preferred_element_type=jnp.float32).astype(jnp.bfloat16)
    wd = wd_ref[0].astype(jnp.bfloat16)
    y_ref[...] = jnp.dot(act, wd,
                         preferred_element_type=jnp.float32) + bd_ref[0]


_mlp = pl.pallas_call(
    _mlp_body,
    grid_spec=pltpu.PrefetchScalarGridSpec(
        num_scalar_prefetch=2,
        grid=(NB,),
        in_specs=[
            pl.BlockSpec((M, H), lambda b, be, bi: (bi[b], 0)),
            pl.BlockSpec((1, H, 2 * I), lambda b, be, bi: (be[b], 0, 0)),
            pl.BlockSpec((1, I, H), lambda b, be, bi: (be[b], 0, 0)),
            pl.BlockSpec((1, 1, 2 * I), lambda b, be, bi: (be[b], 0, 0)),
            pl.BlockSpec((1, 1, H), lambda b, be, bi: (be[b], 0, 0)),
        ],
        out_specs=pl.BlockSpec((M, H), lambda b, be, bi: (bi[b], 0)),
        scratch_shapes=[pltpu.VMEM((2 * I, I), jnp.bfloat16)],
    ),
    out_shape=jax.ShapeDtypeStruct((P, H), jnp.float32),
)


NW = 32
GCH = 16
GNCH = P // NW // GCH
GNBUF = 6


@functools.partial(
    pl.kernel,
    out_type=jax.ShapeDtypeStruct((P, H), jnp.float32),
    mesh=plsc.VectorSubcoreMesh(core_axis_name="c", subcore_axis_name="s"),
    scratch_types=[
        [pltpu.VMEM((GCH,), jnp.int32) for _ in range(GNBUF)],
        [pltpu.VMEM((GCH, H), jnp.float32) for _ in range(GNBUF)],
        [pltpu.SemaphoreType.DMA for _ in range(GNBUF)],
        [pltpu.SemaphoreType.DMA for _ in range(GNBUF)],
    ],
)
def _sc_gather(hs_hbm, tok_hbm, out_hbm, idx_v, rows_v, gsem, wsem):
    wid = lax.axis_index("s") * 2 + lax.axis_index("c")
    base = wid * (P // NW)

    def start_gather(j):
        b = j % GNBUF
        pltpu.sync_copy(tok_hbm.at[pl.ds(base + j * GCH, GCH)], idx_v[b])
        return pltpu.async_copy(hs_hbm.at[idx_v[b]], rows_v[b], gsem[b])

    gops = {j: start_gather(j) for j in range(GNBUF)}
    wops = {}
    for i in range(GNCH):
        b = i % GNBUF
        gops.pop(i).wait()
        wops[i] = pltpu.async_copy(
            rows_v[b], out_hbm.at[pl.ds(base + i * GCH, GCH)], wsem[b])
        j = i + GNBUF
        if j < GNCH:
            wops.pop(i).wait()
            gops[j] = start_gather(j)
    for i in sorted(wops):
        wops.pop(i).wait()


def kernel(hidden_states, router_weight, router_bias, gate_up_proj,
           gate_up_proj_bias, down_proj, down_proj_bias):
    hs = hidden_states.reshape(S, H)
    tw, ti = _router(hs, router_weight, router_bias.reshape(1, E))

    eid = ti.reshape(A)
    sort_idx = jnp.argsort(eid, stable=True).astype(jnp.int32)
    sorted_eid = jnp.take(eid, sort_idx)
    counts = jnp.sum(eid[None, :] == jnp.arange(E, dtype=jnp.int32)[:, None],
                     axis=1).astype(jnp.int32)
    cum_c = jnp.cumsum(counts)
    start = (cum_c - counts).astype(jnp.int32)
    m_e = (counts + M - 1) // M
    cum_m = jnp.cumsum(m_e)
    blkoff = (cum_m - m_e).astype(jnp.int32)
    nb_total = cum_m[-1]
    j = jnp.arange(A, dtype=jnp.int32)
    prow = blkoff[sorted_eid] * M + (j - start[sorted_eid])
    tok_sorted = sort_idx // K
    gather_tok = jnp.zeros(P, jnp.int32).at[prow].set(tok_sorted)
    pos = jnp.zeros(A, jnp.int32).at[sort_idx].set(prow).reshape(S, K)
    barange = jnp.arange(NB, dtype=jnp.int32)
    bidx = jnp.where(barange < nb_total, barange, nb_total - 1).astype(jnp.int32)
    be = jnp.searchsorted(cum_m, bidx, side="right").astype(jnp.int32)

    bgu = gate_up_proj_bias.reshape(E, 1, 2 * I)
    bd = down_proj_bias.reshape(E, 1, H)

    x_sorted = _sc_gather(hs, gather_tok)

    y = _mlp(be, bidx, x_sorted, gate_up_proj, down_proj, bgu, bd)

    out = (tw[:, 0:1] * jnp.take(y, pos[:, 0], axis=0)
           + tw[:, 1:2] * jnp.take(y, pos[:, 1], axis=0))
    return out.reshape(1, S, H), tw.reshape(1, S, 2)


# speedup vs baseline: 1.3291x; 1.3291x over previous
import functools

import jax
import jax.numpy as jnp
from jax import lax
from jax.experimental import pallas as pl
from jax.experimental.pallas import tpu as pltpu
from jax.experimental.pallas import tpu_sc as plsc

S, H, E, I, K = 2048, 1024, 8, 1024, 2
ALPHA, LIMIT = 1.702, 7.0
M = 256
A = S * K
NB = A // M + E
P = NB * M
RB = 256


def _router_body(x_ref, rw_ref, rb_ref, tw_ref, ti_ref):
    x = x_ref[...]
    rw = rw_ref[...]
    logits = lax.dot_general(x, rw, (((1,), (1,)), ((), ())),
                             preferred_element_type=jnp.float32) + rb_ref[...]
    iota = lax.broadcasted_iota(jnp.int32, (RB, E), 1)
    m0 = jnp.max(logits, axis=1, keepdims=True)
    i0 = jnp.min(jnp.where(logits == m0, iota, E), axis=1, keepdims=True)
    masked = jnp.where(iota == i0, -jnp.inf, logits)
    m1 = jnp.max(masked, axis=1, keepdims=True)
    i1 = jnp.min(jnp.where(masked == m1, iota, E), axis=1, keepdims=True)
    e1 = jnp.exp(m1 - m0)
    w0 = 1.0 / (1.0 + e1)
    w1 = e1 * w0
    lane2 = lax.broadcasted_iota(jnp.int32, (RB, 2), 1)
    tw_ref[...] = jnp.where(lane2 == 0, w0, w1)
    ti_ref[...] = jnp.where(lane2 == 0, i0, i1)


_router = pl.pallas_call(
    _router_body,
    grid=(S // RB,),
    in_specs=[
        pl.BlockSpec((RB, H), lambda b: (b, 0)),
        pl.BlockSpec((E, H), lambda b: (0, 0)),
        pl.BlockSpec((1, E), lambda b: (0, 0)),
    ],
    out_specs=[
        pl.BlockSpec((RB, 2), lambda b: (b, 0)),
        pl.BlockSpec((RB, 2), lambda b: (b, 0)),
    ],
    out_shape=[
        jax.ShapeDtypeStruct((S, 2), jnp.float32),
        jax.ShapeDtypeStruct((S, 2), jnp.int32),
    ],
)


def _mlp_body(be_ref, bidx_ref, x_ref, wgu_ref, wd_ref,
              bgu_ref, bd_ref, y_ref, p_ref):
    del be_ref, bidx_ref
    @pl.when(pl.program_id(0) == 0)
    def _build_selector():
        r = lax.broadcasted_iota(jnp.int32, (2 * I, I), 0)
        c = lax.broadcasted_iota(jnp.int32, (2 * I, I), 1)
        p_ref[...] = (r == 2 * c).astype(jnp.bfloat16)

    x = x_ref[...].astype(jnp.bfloat16)
    wgu = wgu_ref[0].astype(jnp.bfloat16)
    gu = jnp.dot(x, wgu, preferred_element_type=jnp.float32) + bgu_ref[0]
    gate = jnp.minimum(gu, LIMIT)
    glu = gate * jax.nn.sigmoid(gate * ALPHA)
    up1 = jnp.clip(gu, -LIMIT, LIMIT) + 1.0
    up1s = pltpu.roll(up1, 2 * I - 1, 1)
    prod = (glu * up1s).astype(jnp.bfloat16)
    act = jnp.dot(prod, p_ref[...],
                  preferred_element_type=jnp.float32).astype(jnp.bfloat16)
    wd = wd_ref[0].astype(jnp.bfloat16)
    y_ref[...] = jnp.dot(act, wd,
                         preferred_element_type=jnp.float32) + bd_ref[0]


_mlp = pl.pallas_call(
    _mlp_body,
    grid_spec=pltpu.PrefetchScalarGridSpec(
        num_scalar_prefetch=2,
        grid=(NB,),
        in_specs=[
            pl.BlockSpec((M, H), lambda b, be, bi: (bi[b], 0)),
            pl.BlockSpec((1, H, 2 * I), lambda b, be, bi: (be[b], 0, 0)),
            pl.BlockSpec((1, I, H), lambda b, be, bi: (be[b], 0, 0)),
            pl.BlockSpec((1, 1, 2 * I), lambda b, be, bi: (be[b], 0, 0)),
            pl.BlockSpec((1, 1, H), lambda b, be, bi: (be[b], 0, 0)),
        ],
        out_specs=pl.BlockSpec((M, H), lambda b, be, bi: (bi[b], 0)),
        scratch_shapes=[pltpu.VMEM((2 * I, I), jnp.bfloat16)],
    ),
    out_shape=jax.ShapeDtypeStruct((P, H), jnp.float32),
)


NW = 32
GCH = 16
GNCH = P // NW // GCH
GNBUF = 6


@functools.partial(
    pl.kernel,
    out_type=jax.ShapeDtypeStruct((P, H), jnp.float32),
    mesh=plsc.VectorSubcoreMesh(core_axis_name="c", subcore_axis_name="s"),
    scratch_types=[
        [pltpu.VMEM((GCH,), jnp.int32) for _ in range(GNBUF)],
        [pltpu.VMEM((GCH, H), jnp.float32) for _ in range(GNBUF)],
        [pltpu.SemaphoreType.DMA for _ in range(GNBUF)],
        [pltpu.SemaphoreType.DMA for _ in range(GNBUF)],
    ],
)
def _sc_gather(hs_hbm, tok_hbm, out_hbm, idx_v, rows_v, gsem, wsem):
    wid = lax.axis_index("s") * 2 + lax.axis_index("c")
    base = wid * (P // NW)

    def start_gather(j):
        b = j % GNBUF
        pltpu.sync_copy(tok_hbm.at[pl.ds(base + j * GCH, GCH)], idx_v[b])
        return pltpu.async_copy(hs_hbm.at[idx_v[b]], rows_v[b], gsem[b])

    gops = {j: start_gather(j) for j in range(GNBUF)}
    wops = {}
    for i in range(GNCH):
        b = i % GNBUF
        gops.pop(i).wait()
        wops[i] = pltpu.async_copy(
            rows_v[b], out_hbm.at[pl.ds(base + i * GCH, GCH)], wsem[b])
        j = i + GNBUF
        if j < GNCH:
            wops.pop(i).wait()
            gops[j] = start_gather(j)
    for i in sorted(wops):
        wops.pop(i).wait()


def kernel(hidden_states, router_weight, router_bias, gate_up_proj,
           gate_up_proj_bias, down_proj, down_proj_bias):
    hs = hidden_states.reshape(S, H)
    tw, ti = _router(hs, router_weight, router_bias.reshape(1, E))

    eid = ti.reshape(A)
    oh = (eid[:, None] == jnp.arange(E, dtype=jnp.int32)[None, :]).astype(
        jnp.int32)
    cum = jnp.cumsum(oh, axis=0)
    counts = cum[-1]
    rank = jnp.sum(oh * (cum - oh), axis=1)
    m_e = (counts + M - 1) // M
    cum_m = jnp.cumsum(m_e)
    blkoff = (cum_m - m_e).astype(jnp.int32)
    nb_total = cum_m[-1]
    prow = jnp.sum(oh * blkoff[None, :], axis=1) * M + rank
    pos = prow.reshape(S, K)
    gather_tok = jnp.zeros(P, jnp.int32).at[prow].set(
        jnp.arange(A, dtype=jnp.int32) // K)
    barange = jnp.arange(NB, dtype=jnp.int32)
    bidx = jnp.where(barange < nb_total, barange, nb_total - 1).astype(jnp.int32)
    be = jnp.sum((cum_m[None, :] <= bidx[:, None]).astype(jnp.int32), axis=1)

    bgu = gate_up_proj_bias.reshape(E, 1, 2 * I)
    bd = down_proj_bias.reshape(E, 1, H)

    x_sorted = jnp.take(hs, gather_tok, axis=0)

    y = _mlp(be, bidx, x_sorted, gate_up_proj, down_proj, bgu, bd)

    out = (tw[:, 0:1] * jnp.take(y, pos[:, 0], axis=0)
           + tw[:, 1:2] * jnp.take(y, pos[:, 1], axis=0))
    return out.reshape(1, S, H), tw.reshape(1, S, 2)


# speedup vs baseline: 1.3696x; 1.0305x over previous
import functools

import jax
import jax.numpy as jnp
from jax import lax
from jax.experimental import pallas as pl
from jax.experimental.pallas import tpu as pltpu
from jax.experimental.pallas import tpu_sc as plsc

S, H, E, I, K = 2048, 1024, 8, 1024, 2
ALPHA, LIMIT = 1.702, 7.0
M = 256
A = S * K
NB = A // M + E
P = NB * M
RB = 256


def _router_body(x_ref, rw_ref, rb_ref, tw_ref, ti_ref):
    x = x_ref[...]
    rw = rw_ref[...]
    logits = lax.dot_general(x, rw, (((1,), (1,)), ((), ())),
                             preferred_element_type=jnp.float32) + rb_ref[...]
    iota = lax.broadcasted_iota(jnp.int32, (RB, E), 1)
    m0 = jnp.max(logits, axis=1, keepdims=True)
    i0 = jnp.min(jnp.where(logits == m0, iota, E), axis=1, keepdims=True)
    masked = jnp.where(iota == i0, -jnp.inf, logits)
    m1 = jnp.max(masked, axis=1, keepdims=True)
    i1 = jnp.min(jnp.where(masked == m1, iota, E), axis=1, keepdims=True)
    e1 = jnp.exp(m1 - m0)
    w0 = 1.0 / (1.0 + e1)
    w1 = e1 * w0
    lane2 = lax.broadcasted_iota(jnp.int32, (RB, 2), 1)
    tw_ref[...] = jnp.where(lane2 == 0, w0, w1)
    ti_ref[...] = jnp.where(lane2 == 0, i0, i1)


_router = pl.pallas_call(
    _router_body,
    grid=(S // RB,),
    in_specs=[
        pl.BlockSpec((RB, H), lambda b: (b, 0)),
        pl.BlockSpec((E, H), lambda b: (0, 0)),
        pl.BlockSpec((1, E), lambda b: (0, 0)),
    ],
    out_specs=[
        pl.BlockSpec((RB, 2), lambda b: (b, 0)),
        pl.BlockSpec((RB, 2), lambda b: (b, 0)),
    ],
    out_shape=[
        jax.ShapeDtypeStruct((S, 2), jnp.float32),
        jax.ShapeDtypeStruct((S, 2), jnp.int32),
    ],
)


def _mlp_body(be_ref, bidx_ref, x_ref, wgu_ref, wd_ref,
              bgu_ref, bd_ref, y_ref, p_ref):
    del be_ref, bidx_ref
    @pl.when(pl.program_id(0) == 0)
    def _build_selector():
        r = lax.broadcasted_iota(jnp.int32, (2 * I, I), 0)
        c = lax.broadcasted_iota(jnp.int32, (2 * I, I), 1)
        p_ref[...] = (r == 2 * c).astype(jnp.bfloat16)

    x = x_ref[...]
    wgu = wgu_ref[0].astype(jnp.bfloat16)
    gu = jnp.dot(x, wgu, preferred_element_type=jnp.float32) + bgu_ref[0]
    gate = jnp.minimum(gu, LIMIT)
    glu = gate * jax.nn.sigmoid(gate * ALPHA)
    up1 = jnp.clip(gu, -LIMIT, LIMIT) + 1.0
    up1s = pltpu.roll(up1, 2 * I - 1, 1)
    prod = (glu * up1s).astype(jnp.bfloat16)
    act = jnp.dot(prod, p_ref[...],
                  preferred_element_type=jnp.float32).astype(jnp.bfloat16)
    wd = wd_ref[0].astype(jnp.bfloat16)
    y_ref[...] = jnp.dot(act, wd,
                         preferred_element_type=jnp.float32) + bd_ref[0]


_mlp = pl.pallas_call(
    _mlp_body,
    grid_spec=pltpu.PrefetchScalarGridSpec(
        num_scalar_prefetch=2,
        grid=(NB,),
        in_specs=[
            pl.BlockSpec((M, H), lambda b, be, bi: (bi[b], 0)),
            pl.BlockSpec((1, H, 2 * I), lambda b, be, bi: (be[b], 0, 0)),
            pl.BlockSpec((1, I, H), lambda b, be, bi: (be[b], 0, 0)),
            pl.BlockSpec((1, 1, 2 * I), lambda b, be, bi: (be[b], 0, 0)),
            pl.BlockSpec((1, 1, H), lambda b, be, bi: (be[b], 0, 0)),
        ],
        out_specs=pl.BlockSpec((M, H), lambda b, be, bi: (bi[b], 0)),
        scratch_shapes=[pltpu.VMEM((2 * I, I), jnp.bfloat16)],
    ),
    out_shape=jax.ShapeDtypeStruct((P, H), jnp.float32),
)


def kernel(hidden_states, router_weight, router_bias, gate_up_proj,
           gate_up_proj_bias, down_proj, down_proj_bias):
    hs = hidden_states.reshape(S, H)
    tw, ti = _router(hs, router_weight, router_bias.reshape(1, E))

    eid = ti.reshape(A)
    oh = (eid[:, None] == jnp.arange(E, dtype=jnp.int32)[None, :]).astype(
        jnp.int32)
    cum = jnp.cumsum(oh, axis=0)
    counts = cum[-1]
    rank = jnp.sum(oh * (cum - oh), axis=1)
    m_e = (counts + M - 1) // M
    cum_m = jnp.cumsum(m_e)
    blkoff = (cum_m - m_e).astype(jnp.int32)
    nb_total = cum_m[-1]
    prow = jnp.sum(oh * blkoff[None, :], axis=1) * M + rank
    pos = prow.reshape(S, K)
    gather_tok = jnp.zeros(P, jnp.int32).at[prow].set(
        jnp.arange(A, dtype=jnp.int32) // K)
    barange = jnp.arange(NB, dtype=jnp.int32)
    bidx = jnp.where(barange < nb_total, barange, nb_total - 1).astype(jnp.int32)
    be = jnp.sum((cum_m[None, :] <= bidx[:, None]).astype(jnp.int32), axis=1)

    bgu = gate_up_proj_bias.reshape(E, 1, 2 * I)
    bd = down_proj_bias.reshape(E, 1, H)

    x_sorted = jnp.take(hs.astype(jnp.bfloat16), gather_tok, axis=0)

    y = _mlp(be, bidx, x_sorted, gate_up_proj, down_proj, bgu, bd)

    out = (tw[:, 0:1] * jnp.take(y, pos[:, 0], axis=0)
           + tw[:, 1:2] * jnp.take(y, pos[:, 1], axis=0))
    return out.reshape(1, S, H), tw.reshape(1, S, 2)
